# baseline (speedup 1.0000x reference)
import numpy as np
import jax
import jax.numpy as jnp
from jax import lax
from jax.experimental import pallas as pl
from jax.experimental.pallas import tpu as pltpu
from jax.experimental.pallas import tpu_sc as plsc

HIDDEN = 768
NUM_HEADS = 12
NUM_KV_HEADS = 4
HEAD_DIM = 64
HALF = HEAD_DIM // 2
NUM_EXPERTS = 64
TOP_K = 8
ROPE_THETA = 10000.0
T = 2048
Q_SIZE = NUM_HEADS * HEAD_DIM
KV_SIZE = NUM_KV_HEADS * HEAD_DIM
SCALING = HEAD_DIM ** -0.5
EPS = 1e-6
REP = NUM_HEADS // NUM_KV_HEADS

TB = 256
QB = 256

SC_NC = 2
SC_NS = 16
NW = SC_NC * SC_NS
N_PERM = T * TOP_K
ROWS_PER_W = N_PERM // NW
GCHUNK = 64


def _qkv_body(pos_ref, x_ref, wpre_ref, wqkv_ref, q_ref, k_ref, v_ref):
    x = x_ref[...]
    var = jnp.mean(x * x, axis=-1, keepdims=True)
    h = x * lax.rsqrt(var + EPS) * wpre_ref[...]
    qkv = lax.dot_general(h, wqkv_ref[...], (((1,), (1,)), ((), ())),
                          preferred_element_type=jnp.float32)
    pos = pos_ref[...]
    expo = lax.broadcasted_iota(jnp.int32, (1, HALF), 1).astype(jnp.float32) * (2.0 / HEAD_DIM)
    inv_freq = jnp.exp(expo * (-float(np.log(ROPE_THETA))))
    freqs = pos * inv_freq
    c = jnp.cos(freqs)
    s = jnp.sin(freqs)

    def rot(x):
        x1 = x[:, :HALF]
        x2 = x[:, HALF:]
        return jnp.concatenate([x1 * c - x2 * s, x2 * c + x1 * s], axis=1)

    for hh in range(NUM_HEADS):
        sl = slice(hh * HEAD_DIM, (hh + 1) * HEAD_DIM)
        q_ref[:, sl] = (rot(qkv[:, sl]) * SCALING).astype(jnp.bfloat16)
    for kh in range(NUM_KV_HEADS):
        sl = slice(Q_SIZE + kh * HEAD_DIM, Q_SIZE + (kh + 1) * HEAD_DIM)
        k_ref[:, kh * HEAD_DIM:(kh + 1) * HEAD_DIM] = rot(qkv[:, sl]).astype(jnp.bfloat16)
    v_ref[...] = qkv[:, Q_SIZE + KV_SIZE:].astype(jnp.bfloat16)


def _attn_body(q_ref, k_ref, v_ref, o_ref):
    qb = pl.program_id(0)
    row = qb * QB + lax.broadcasted_iota(jnp.int32, (QB, T), 0)
    col = lax.broadcasted_iota(jnp.int32, (QB, T), 1)
    causal = row >= col
    qs = q_ref[...]
    ks = k_ref[...]
    vs = v_ref[...]
    for h in range(NUM_HEADS):
        kh = h // REP
        q = qs[:, h * HEAD_DIM:(h + 1) * HEAD_DIM]
        k = ks[:, kh * HEAD_DIM:(kh + 1) * HEAD_DIM]
        v = vs[:, kh * HEAD_DIM:(kh + 1) * HEAD_DIM]
        s = lax.dot_general(q, k, (((1,), (1,)), ((), ())),
                            preferred_element_type=jnp.float32)
        s = jnp.where(causal, s, jnp.finfo(jnp.float32).min)
        m = jnp.max(s, axis=-1, keepdims=True)
        p = jnp.exp(s - m)
        l = jnp.sum(p, axis=-1, keepdims=True)
        o = lax.dot_general(p.astype(jnp.bfloat16), v, (((1,), (0,)), ((), ())),
                            preferred_element_type=jnp.float32)
        o_ref[:, h * HEAD_DIM:(h + 1) * HEAD_DIM] = o / l


def _oproj_body(a_ref, wo_ref, res_ref, wpost_ref, out_ref):
    o = lax.dot_general(a_ref[...], wo_ref[...], (((1,), (1,)), ((), ())),
                        preferred_element_type=jnp.float32)
    r = o + res_ref[...]
    var = jnp.mean(r * r, axis=-1, keepdims=True)
    out_ref[...] = r * lax.rsqrt(var + EPS) * wpost_ref[...]


def _topk_body(el_ref, tw_ref, ti_ref):
    x = el_ref[...]
    iota = lax.broadcasted_iota(jnp.int32, (T, NUM_EXPERTS), 1)
    vals = []
    ids = []
    for _ in range(TOP_K):
        m = jnp.max(x, axis=1, keepdims=True)
        eq = x == m
        idx = jnp.min(jnp.where(eq, iota, NUM_EXPERTS), axis=1, keepdims=True)
        vals.append(m)
        ids.append(idx)
        x = jnp.where(iota == idx, -1.0, x)
    tw = jnp.concatenate(vals, axis=1)
    ti = jnp.concatenate(ids, axis=1)
    tw_ref[...] = tw / jnp.sum(tw, axis=1, keepdims=True)
    ti_ref[...] = ti


SB = 512
NSB = N_PERM // SB
OMAX = 512


def _sort_body(flat_ref, reorder_ref, src_ref):
    iota_e = lax.broadcasted_iota(jnp.int32, (SB, NUM_EXPERTS), 1)
    r_i = lax.broadcasted_iota(jnp.int32, (SB, SB), 0)
    c_i = lax.broadcasted_iota(jnp.int32, (SB, SB), 1)
    tril_incl = (c_i <= r_i).astype(jnp.float32)
    iota_o = lax.broadcasted_iota(jnp.int32, (SB, OMAX), 1).astype(jnp.float32)

    def onehot(b):
        fb = flat_ref[pl.ds(b * SB, SB), :]
        return (fb == iota_e).astype(jnp.float32)

    occ_incl = []
    tots = []
    for b in range(NSB):
        oneh = onehot(b)
        pref = lax.dot_general(tril_incl, oneh, (((1,), (0,)), ((), ())),
                               preferred_element_type=jnp.float32)
        occ_incl.append(jnp.sum(pref * oneh, axis=1, keepdims=True))
        tots.append(pref[SB - 1:SB, :])
    tots = jnp.concatenate(tots, axis=0)
    rb = lax.broadcasted_iota(jnp.int32, (NSB, NSB), 0)
    cb = lax.broadcasted_iota(jnp.int32, (NSB, NSB), 1)
    tril_sb = (cb < rb).astype(jnp.float32)
    blockoff = lax.dot_general(tril_sb, tots, (((1,), (0,)), ((), ())),
                               preferred_element_type=jnp.float32)
    cnt = jnp.sum(tots, axis=0, keepdims=True)
    re_ = lax.broadcasted_iota(jnp.int32, (NUM_EXPERTS, NUM_EXPERTS), 0)
    ce_ = lax.broadcasted_iota(jnp.int32, (NUM_EXPERTS, NUM_EXPERTS), 1)
    triu_e = (re_ < ce_).astype(jnp.float32)
    offsets = lax.dot_general(cnt, triu_e, (((1,), (0,)), ((), ())),
                              preferred_element_type=jnp.float32)
    offsets_incl = offsets + cnt

    s_tab = jnp.zeros((NUM_EXPERTS, OMAX), dtype=jnp.float32)
    for b in range(NSB):
        oneh = onehot(b)
        occ_g = occ_incl[b] - 1.0 + jnp.sum(oneh * blockoff[b:b + 1, :],
                                            axis=1, keepdims=True)
        oneho = (occ_g == iota_o).astype(jnp.float32)
        ivals = (b * SB + lax.broadcasted_iota(jnp.int32, (SB, 1), 0)).astype(jnp.float32)
        s_tab = s_tab + lax.dot_general(oneh, oneho * ivals,
                                        (((0,), (0,)), ((), ())),
                                        preferred_element_type=jnp.float32)

    for b in range(NSB):
        pvec = (b * SB + lax.broadcasted_iota(jnp.int32, (SB, 1), 0)).astype(jnp.float32)
        ge_lo = (pvec >= offsets).astype(jnp.float32)
        ge_hi = (pvec >= offsets_incl).astype(jnp.float32)
        oneh_e = ge_lo - ge_hi
        o_p = pvec - jnp.sum(oneh_e * offsets, axis=1, keepdims=True)
        tmp = lax.dot_general(oneh_e, s_tab, (((1,), (0,)), ((), ())),
                              preferred_element_type=jnp.float32)
        oneho_p = (o_p == iota_o).astype(jnp.float32)
        r_p = jnp.sum(tmp * oneho_p, axis=1, keepdims=True)
        ri = r_p.astype(jnp.int32)
        reorder_ref[pl.ds(b * SB, SB), :] = ri
        src_ref[pl.ds(b * SB, SB), :] = lax.shift_right_logical(ri, 3)


def _sc_gather_body(table_hbm, idx_hbm, out_hbm, idx_v, buf0, buf1, sem0, sem1):
    wid = lax.axis_index("s") * SC_NC + lax.axis_index("c")
    base = wid * ROWS_PER_W
    pltpu.sync_copy(idx_hbm.at[pl.ds(base, ROWS_PER_W)], idx_v)
    nchunk = ROWS_PER_W // GCHUNK
    bufs = (buf0, buf1)
    sems = (sem0, sem1)
    handles = [None, None]
    handles[0] = pltpu.async_copy(
        table_hbm.at[idx_v.at[pl.ds(0, GCHUNK)]], bufs[0], sems[0])
    for c in range(1, nchunk + 1):
        if c < nchunk:
            b = c % 2
            handles[b] = pltpu.async_copy(
                table_hbm.at[idx_v.at[pl.ds(c * GCHUNK, GCHUNK)]], bufs[b], sems[b])
        pb = (c - 1) % 2
        handles[pb].wait()
        pltpu.sync_copy(bufs[pb], out_hbm.at[pl.ds(base + (c - 1) * GCHUNK, GCHUNK)])


def _permute_gather_sc(out, src_idx):
    mesh = plsc.VectorSubcoreMesh(core_axis_name="c", subcore_axis_name="s")
    kfn = pl.kernel(
        _sc_gather_body,
        out_type=jax.ShapeDtypeStruct((N_PERM, HIDDEN), jnp.float32),
        mesh=mesh,
        scratch_types=[
            pltpu.VMEM((ROWS_PER_W,), jnp.int32),
            pltpu.VMEM((GCHUNK, HIDDEN), jnp.float32),
            pltpu.VMEM((GCHUNK, HIDDEN), jnp.float32),
            pltpu.SemaphoreType.DMA,
            pltpu.SemaphoreType.DMA,
        ],
    )
    return kfn(out, src_idx)


def kernel(positions, hidden_states, kv_cache, w_pre, W_qkv, W_o, w_post, W_gate):
    pos2d = positions.astype(jnp.float32).reshape(T, 1)
    wpre2d = w_pre.reshape(1, HIDDEN)
    wpost2d = w_post.reshape(1, HIDDEN)

    q, k, v = pl.pallas_call(
        _qkv_body,
        grid=(T // TB,),
        in_specs=[
            pl.BlockSpec((TB, 1), lambda i: (i, 0)),
            pl.BlockSpec((TB, HIDDEN), lambda i: (i, 0)),
            pl.BlockSpec((1, HIDDEN), lambda i: (0, 0)),
            pl.BlockSpec((Q_SIZE + 2 * KV_SIZE, HIDDEN), lambda i: (0, 0)),
        ],
        out_specs=[
            pl.BlockSpec((TB, Q_SIZE), lambda i: (i, 0)),
            pl.BlockSpec((TB, KV_SIZE), lambda i: (i, 0)),
            pl.BlockSpec((TB, KV_SIZE), lambda i: (i, 0)),
        ],
        out_shape=[
            jax.ShapeDtypeStruct((T, Q_SIZE), jnp.bfloat16),
            jax.ShapeDtypeStruct((T, KV_SIZE), jnp.bfloat16),
            jax.ShapeDtypeStruct((T, KV_SIZE), jnp.bfloat16),
        ],
    )(pos2d, hidden_states, wpre2d, W_qkv)

    attn = pl.pallas_call(
        _attn_body,
        grid=(T // QB,),
        in_specs=[
            pl.BlockSpec((QB, Q_SIZE), lambda qb: (qb, 0)),
            pl.BlockSpec((T, KV_SIZE), lambda qb: (0, 0)),
            pl.BlockSpec((T, KV_SIZE), lambda qb: (0, 0)),
        ],
        out_specs=pl.BlockSpec((QB, Q_SIZE), lambda qb: (qb, 0)),
        out_shape=jax.ShapeDtypeStruct((T, Q_SIZE), jnp.float32),
    )(q, k, v)

    out = pl.pallas_call(
        _oproj_body,
        grid=(T // TB,),
        in_specs=[
            pl.BlockSpec((TB, Q_SIZE), lambda i: (i, 0)),
            pl.BlockSpec((HIDDEN, Q_SIZE), lambda i: (0, 0)),
            pl.BlockSpec((TB, HIDDEN), lambda i: (i, 0)),
            pl.BlockSpec((1, HIDDEN), lambda i: (0, 0)),
        ],
        out_specs=pl.BlockSpec((TB, HIDDEN), lambda i: (i, 0)),
        out_shape=jax.ShapeDtypeStruct((T, HIDDEN), jnp.float32),
    )(attn, W_o, hidden_states, wpost2d)

    key_r = jax.random.fold_in(jax.random.key(0), 123)
    expert_logits = jax.random.uniform(key_r, (T, NUM_EXPERTS), dtype=jnp.float32)
    topk_weights, topk_ids = pl.pallas_call(
        _topk_body,
        out_shape=[
            jax.ShapeDtypeStruct((T, TOP_K), jnp.float32),
            jax.ShapeDtypeStruct((T, TOP_K), jnp.int32),
        ],
    )(expert_logits)
    flat2 = topk_ids.reshape(N_PERM, 1)
    reorder2, src2 = pl.pallas_call(
        _sort_body,
        out_shape=[
            jax.ShapeDtypeStruct((N_PERM, 1), jnp.int32),
            jax.ShapeDtypeStruct((N_PERM, 1), jnp.int32),
        ],
    )(flat2)
    reorder_ids = reorder2.reshape(-1)
    src_idx = src2.reshape(-1)

    permuted_output = _permute_gather_sc(out, src_idx)
    return (permuted_output, topk_weights, topk_ids, reorder_ids)


# speedup vs baseline: 1.0520x; 1.0520x over previous
import numpy as np
import jax
import jax.numpy as jnp
from jax import lax
from jax.experimental import pallas as pl
from jax.experimental.pallas import tpu as pltpu
from jax.experimental.pallas import tpu_sc as plsc

HIDDEN = 768
NUM_HEADS = 12
NUM_KV_HEADS = 4
HEAD_DIM = 64
HALF = HEAD_DIM // 2
NUM_EXPERTS = 64
TOP_K = 8
ROPE_THETA = 10000.0
T = 2048
Q_SIZE = NUM_HEADS * HEAD_DIM
KV_SIZE = NUM_KV_HEADS * HEAD_DIM
SCALING = HEAD_DIM ** -0.5
EPS = 1e-6
REP = NUM_HEADS // NUM_KV_HEADS

TB = 256
QB = 256

SC_NC = 2
SC_NS = 16
NW = SC_NC * SC_NS
N_PERM = T * TOP_K
ROWS_PER_W = N_PERM // NW
GCHUNK = 64


def _qkv_body(pos_ref, x_ref, wpre_ref, wqkv_ref, q_ref, k_ref, v_ref):
    x = x_ref[...]
    var = jnp.mean(x * x, axis=-1, keepdims=True)
    h = x * lax.rsqrt(var + EPS) * wpre_ref[...]
    qkv = lax.dot_general(h, wqkv_ref[...], (((1,), (1,)), ((), ())),
                          preferred_element_type=jnp.float32)
    pos = pos_ref[...]
    expo = lax.broadcasted_iota(jnp.int32, (1, HALF), 1).astype(jnp.float32) * (2.0 / HEAD_DIM)
    inv_freq = jnp.exp(expo * (-float(np.log(ROPE_THETA))))
    freqs = pos * inv_freq
    c = jnp.cos(freqs)
    s = jnp.sin(freqs)

    def rot(x):
        x1 = x[:, :HALF]
        x2 = x[:, HALF:]
        return jnp.concatenate([x1 * c - x2 * s, x2 * c + x1 * s], axis=1)

    for hh in range(NUM_HEADS):
        sl = slice(hh * HEAD_DIM, (hh + 1) * HEAD_DIM)
        q_ref[:, sl] = (rot(qkv[:, sl]) * SCALING).astype(jnp.bfloat16)
    for kh in range(NUM_KV_HEADS):
        sl = slice(Q_SIZE + kh * HEAD_DIM, Q_SIZE + (kh + 1) * HEAD_DIM)
        k_ref[:, kh * HEAD_DIM:(kh + 1) * HEAD_DIM] = rot(qkv[:, sl]).astype(jnp.bfloat16)
    v_ref[...] = qkv[:, Q_SIZE + KV_SIZE:].astype(jnp.bfloat16)


def _attn_body(q_ref, k_ref, v_ref, o_ref):
    qb = pl.program_id(0)
    row = qb * QB + lax.broadcasted_iota(jnp.int32, (QB, T), 0)
    col = lax.broadcasted_iota(jnp.int32, (QB, T), 1)
    causal = row >= col
    qs = q_ref[...]
    ks = k_ref[...]
    vs = v_ref[...]
    for h in range(NUM_HEADS):
        kh = h // REP
        q = qs[:, h * HEAD_DIM:(h + 1) * HEAD_DIM]
        k = ks[:, kh * HEAD_DIM:(kh + 1) * HEAD_DIM]
        v = vs[:, kh * HEAD_DIM:(kh + 1) * HEAD_DIM]
        s = lax.dot_general(q, k, (((1,), (1,)), ((), ())),
                            preferred_element_type=jnp.float32)
        s = jnp.where(causal, s, jnp.finfo(jnp.float32).min)
        m = jnp.max(s, axis=-1, keepdims=True)
        p = jnp.exp(s - m)
        l = jnp.sum(p, axis=-1, keepdims=True)
        o = lax.dot_general(p.astype(jnp.bfloat16), v, (((1,), (0,)), ((), ())),
                            preferred_element_type=jnp.float32)
        o_ref[:, h * HEAD_DIM:(h + 1) * HEAD_DIM] = o / l


def _oproj_body(a_ref, wo_ref, res_ref, wpost_ref, out_ref):
    o = lax.dot_general(a_ref[...], wo_ref[...], (((1,), (1,)), ((), ())),
                        preferred_element_type=jnp.float32)
    r = o + res_ref[...]
    var = jnp.mean(r * r, axis=-1, keepdims=True)
    out_ref[...] = r * lax.rsqrt(var + EPS) * wpost_ref[...]


def _topk_body(el_ref, tw_ref, ti_ref):
    x = el_ref[...]
    iota = lax.broadcasted_iota(jnp.int32, (T, NUM_EXPERTS), 1)
    vals = []
    ids = []
    for _ in range(TOP_K):
        m = jnp.max(x, axis=1, keepdims=True)
        eq = x == m
        idx = jnp.min(jnp.where(eq, iota, NUM_EXPERTS), axis=1, keepdims=True)
        vals.append(m)
        ids.append(idx)
        x = jnp.where(iota == idx, -1.0, x)
    tw = jnp.concatenate(vals, axis=1)
    ti = jnp.concatenate(ids, axis=1)
    tw_ref[...] = tw / jnp.sum(tw, axis=1, keepdims=True)
    ti_ref[...] = ti


SB = 512
NSB = N_PERM // SB
OMAX = 384


def _rnd(x):
    return jnp.floor(x + 0.5)


def _sort_body(flat_ref, reorder_ref, src_ref):
    iota_e = lax.broadcasted_iota(jnp.int32, (SB, NUM_EXPERTS), 1)
    r_i = lax.broadcasted_iota(jnp.int32, (SB, SB), 0)
    c_i = lax.broadcasted_iota(jnp.int32, (SB, SB), 1)
    tril_incl = (c_i <= r_i).astype(jnp.bfloat16)
    iota_o = lax.broadcasted_iota(jnp.int32, (SB, OMAX), 1).astype(jnp.float32)

    def onehot(b):
        fb = flat_ref[pl.ds(b * SB, SB), :]
        return (fb == iota_e).astype(jnp.bfloat16)

    occ_incl = []
    tots = []
    for b in range(NSB):
        oneh = onehot(b)
        pref = _rnd(lax.dot_general(tril_incl, oneh, (((1,), (0,)), ((), ())),
                                    preferred_element_type=jnp.float32))
        occ_incl.append(jnp.sum(pref * oneh.astype(jnp.float32),
                                axis=1, keepdims=True))
        tots.append(pref[SB - 1:SB, :])
    tots = jnp.concatenate(tots, axis=0)
    rb = lax.broadcasted_iota(jnp.int32, (NSB, NSB), 0)
    cb = lax.broadcasted_iota(jnp.int32, (NSB, NSB), 1)
    tril_sb = (cb < rb).astype(jnp.float32)
    blockoff = _rnd(lax.dot_general(tril_sb, tots, (((1,), (0,)), ((), ())),
                                    preferred_element_type=jnp.float32))
    cnt = jnp.sum(tots, axis=0, keepdims=True)
    re_ = lax.broadcasted_iota(jnp.int32, (NUM_EXPERTS, NUM_EXPERTS), 0)
    ce_ = lax.broadcasted_iota(jnp.int32, (NUM_EXPERTS, NUM_EXPERTS), 1)
    triu_e = (re_ < ce_).astype(jnp.float32)
    offsets = _rnd(lax.dot_general(cnt, triu_e, (((1,), (0,)), ((), ())),
                                   preferred_element_type=jnp.float32))
    offsets_incl = offsets + cnt

    s_hi = jnp.zeros((NUM_EXPERTS, OMAX), dtype=jnp.float32)
    s_lo = jnp.zeros((NUM_EXPERTS, OMAX), dtype=jnp.float32)
    for b in range(NSB):
        oneh = onehot(b)
        occ_g = occ_incl[b] - 1.0 + jnp.sum(oneh.astype(jnp.float32)
                                            * blockoff[b:b + 1, :],
                                            axis=1, keepdims=True)
        oneho = (occ_g == iota_o).astype(jnp.float32)
        ivals = b * SB + lax.broadcasted_iota(jnp.int32, (SB, 1), 0)
        ihi = (ivals // 128).astype(jnp.float32)
        ilo = (ivals % 128).astype(jnp.float32)
        zhi = (oneho * ihi).astype(jnp.bfloat16)
        zlo = (oneho * ilo).astype(jnp.bfloat16)
        s_hi = s_hi + lax.dot_general(oneh, zhi, (((0,), (0,)), ((), ())),
                                      preferred_element_type=jnp.float32)
        s_lo = s_lo + lax.dot_general(oneh, zlo, (((0,), (0,)), ((), ())),
                                      preferred_element_type=jnp.float32)
    s_hi = _rnd(s_hi).astype(jnp.bfloat16)
    s_lo = _rnd(s_lo).astype(jnp.bfloat16)

    for b in range(NSB):
        pvec = (b * SB + lax.broadcasted_iota(jnp.int32, (SB, 1), 0)).astype(jnp.float32)
        ge_lo = (pvec >= offsets).astype(jnp.float32)
        ge_hi = (pvec >= offsets_incl).astype(jnp.float32)
        oneh_e = ge_lo - ge_hi
        o_p = pvec - jnp.sum(oneh_e * offsets, axis=1, keepdims=True)
        oneh_eb = oneh_e.astype(jnp.bfloat16)
        t_hi = lax.dot_general(oneh_eb, s_hi, (((1,), (0,)), ((), ())),
                               preferred_element_type=jnp.float32)
        t_lo = lax.dot_general(oneh_eb, s_lo, (((1,), (0,)), ((), ())),
                               preferred_element_type=jnp.float32)
        oneho_p = (o_p == iota_o).astype(jnp.float32)
        r_p = jnp.sum((t_hi * 128.0 + t_lo) * oneho_p, axis=1, keepdims=True)
        ri = _rnd(r_p).astype(jnp.int32)
        reorder_ref[pl.ds(b * SB, SB), :] = ri
        src_ref[pl.ds(b * SB, SB), :] = lax.shift_right_logical(ri, 3)


def _sc_gather_body(table_hbm, idx_hbm, out_hbm, idx_v, buf0, buf1, sem0, sem1):
    wid = lax.axis_index("s") * SC_NC + lax.axis_index("c")
    base = wid * ROWS_PER_W
    pltpu.sync_copy(idx_hbm.at[pl.ds(base, ROWS_PER_W)], idx_v)
    nchunk = ROWS_PER_W // GCHUNK
    bufs = (buf0, buf1)
    sems = (sem0, sem1)
    handles = [None, None]
    handles[0] = pltpu.async_copy(
        table_hbm.at[idx_v.at[pl.ds(0, GCHUNK)]], bufs[0], sems[0])
    for c in range(1, nchunk + 1):
        if c < nchunk:
            b = c % 2
            handles[b] = pltpu.async_copy(
                table_hbm.at[idx_v.at[pl.ds(c * GCHUNK, GCHUNK)]], bufs[b], sems[b])
        pb = (c - 1) % 2
        handles[pb].wait()
        pltpu.sync_copy(bufs[pb], out_hbm.at[pl.ds(base + (c - 1) * GCHUNK, GCHUNK)])


def _permute_gather_sc(out, src_idx):
    mesh = plsc.VectorSubcoreMesh(core_axis_name="c", subcore_axis_name="s")
    kfn = pl.kernel(
        _sc_gather_body,
        out_type=jax.ShapeDtypeStruct((N_PERM, HIDDEN), jnp.float32),
        mesh=mesh,
        scratch_types=[
            pltpu.VMEM((ROWS_PER_W,), jnp.int32),
            pltpu.VMEM((GCHUNK, HIDDEN), jnp.float32),
            pltpu.VMEM((GCHUNK, HIDDEN), jnp.float32),
            pltpu.SemaphoreType.DMA,
            pltpu.SemaphoreType.DMA,
        ],
    )
    return kfn(out, src_idx)


def kernel(positions, hidden_states, kv_cache, w_pre, W_qkv, W_o, w_post, W_gate):
    pos2d = positions.astype(jnp.float32).reshape(T, 1)
    wpre2d = w_pre.reshape(1, HIDDEN)
    wpost2d = w_post.reshape(1, HIDDEN)

    q, k, v = pl.pallas_call(
        _qkv_body,
        grid=(T // TB,),
        in_specs=[
            pl.BlockSpec((TB, 1), lambda i: (i, 0)),
            pl.BlockSpec((TB, HIDDEN), lambda i: (i, 0)),
            pl.BlockSpec((1, HIDDEN), lambda i: (0, 0)),
            pl.BlockSpec((Q_SIZE + 2 * KV_SIZE, HIDDEN), lambda i: (0, 0)),
        ],
        out_specs=[
            pl.BlockSpec((TB, Q_SIZE), lambda i: (i, 0)),
            pl.BlockSpec((TB, KV_SIZE), lambda i: (i, 0)),
            pl.BlockSpec((TB, KV_SIZE), lambda i: (i, 0)),
        ],
        out_shape=[
            jax.ShapeDtypeStruct((T, Q_SIZE), jnp.bfloat16),
            jax.ShapeDtypeStruct((T, KV_SIZE), jnp.bfloat16),
            jax.ShapeDtypeStruct((T, KV_SIZE), jnp.bfloat16),
        ],
    )(pos2d, hidden_states, wpre2d, W_qkv)

    attn = pl.pallas_call(
        _attn_body,
        grid=(T // QB,),
        in_specs=[
            pl.BlockSpec((QB, Q_SIZE), lambda qb: (qb, 0)),
            pl.BlockSpec((T, KV_SIZE), lambda qb: (0, 0)),
            pl.BlockSpec((T, KV_SIZE), lambda qb: (0, 0)),
        ],
        out_specs=pl.BlockSpec((QB, Q_SIZE), lambda qb: (qb, 0)),
        out_shape=jax.ShapeDtypeStruct((T, Q_SIZE), jnp.float32),
    )(q, k, v)

    out = pl.pallas_call(
        _oproj_body,
        grid=(T // TB,),
        in_specs=[
            pl.BlockSpec((TB, Q_SIZE), lambda i: (i, 0)),
            pl.BlockSpec((HIDDEN, Q_SIZE), lambda i: (0, 0)),
            pl.BlockSpec((TB, HIDDEN), lambda i: (i, 0)),
            pl.BlockSpec((1, HIDDEN), lambda i: (0, 0)),
        ],
        out_specs=pl.BlockSpec((TB, HIDDEN), lambda i: (i, 0)),
        out_shape=jax.ShapeDtypeStruct((T, HIDDEN), jnp.float32),
    )(attn, W_o, hidden_states, wpost2d)

    key_r = jax.random.fold_in(jax.random.key(0), 123)
    expert_logits = jax.random.uniform(key_r, (T, NUM_EXPERTS), dtype=jnp.float32)
    topk_weights, topk_ids = pl.pallas_call(
        _topk_body,
        out_shape=[
            jax.ShapeDtypeStruct((T, TOP_K), jnp.float32),
            jax.ShapeDtypeStruct((T, TOP_K), jnp.int32),
        ],
    )(expert_logits)
    flat2 = topk_ids.reshape(N_PERM, 1)
    reorder2, src2 = pl.pallas_call(
        _sort_body,
        out_shape=[
            jax.ShapeDtypeStruct((N_PERM, 1), jnp.int32),
            jax.ShapeDtypeStruct((N_PERM, 1), jnp.int32),
        ],
    )(flat2)
    reorder_ids = reorder2.reshape(-1)
    src_idx = src2.reshape(-1)

    permuted_output = _permute_gather_sc(out, src_idx)
    return (permuted_output, topk_weights, topk_ids, reorder_ids)


# speedup vs baseline: 1.1593x; 1.1021x over previous
import numpy as np
import jax
import jax.numpy as jnp
from jax import lax
from jax.experimental import pallas as pl
from jax.experimental.pallas import tpu as pltpu
from jax.experimental.pallas import tpu_sc as plsc

HIDDEN = 768
NUM_HEADS = 12
NUM_KV_HEADS = 4
HEAD_DIM = 64
HALF = HEAD_DIM // 2
NUM_EXPERTS = 64
TOP_K = 8
ROPE_THETA = 10000.0
T = 2048
Q_SIZE = NUM_HEADS * HEAD_DIM
KV_SIZE = NUM_KV_HEADS * HEAD_DIM
SCALING = HEAD_DIM ** -0.5
EPS = 1e-6
REP = NUM_HEADS // NUM_KV_HEADS

TB = 256
QB = 256

SC_NC = 2
SC_NS = 16
NW = SC_NC * SC_NS
N_PERM = T * TOP_K
ROWS_PER_W = N_PERM // NW
GCHUNK = 64


def _qkv_body(pos_ref, x_ref, wpre_ref, wqkv_ref, q_ref, k_ref, v_ref):
    x = x_ref[...]
    var = jnp.mean(x * x, axis=-1, keepdims=True)
    h = x * lax.rsqrt(var + EPS) * wpre_ref[...]
    qkv = lax.dot_general(h, wqkv_ref[...], (((1,), (1,)), ((), ())),
                          preferred_element_type=jnp.float32)
    pos = pos_ref[...]
    expo = lax.broadcasted_iota(jnp.int32, (1, HALF), 1).astype(jnp.float32) * (2.0 / HEAD_DIM)
    inv_freq = jnp.exp(expo * (-float(np.log(ROPE_THETA))))
    freqs = pos * inv_freq
    c = jnp.cos(freqs)
    s = jnp.sin(freqs)

    def rot(x):
        x1 = x[:, :HALF]
        x2 = x[:, HALF:]
        return jnp.concatenate([x1 * c - x2 * s, x2 * c + x1 * s], axis=1)

    for hh in range(NUM_HEADS):
        sl = slice(hh * HEAD_DIM, (hh + 1) * HEAD_DIM)
        q_ref[:, sl] = (rot(qkv[:, sl]) * SCALING).astype(jnp.bfloat16)
    for kh in range(NUM_KV_HEADS):
        sl = slice(Q_SIZE + kh * HEAD_DIM, Q_SIZE + (kh + 1) * HEAD_DIM)
        k_ref[:, kh * HEAD_DIM:(kh + 1) * HEAD_DIM] = rot(qkv[:, sl]).astype(jnp.bfloat16)
    v_ref[...] = qkv[:, Q_SIZE + KV_SIZE:].astype(jnp.bfloat16)


def _attn_body(q_ref, k_ref, v_ref, o_ref):
    qb = pl.program_id(0)
    row = qb * QB + lax.broadcasted_iota(jnp.int32, (QB, T), 0)
    col = lax.broadcasted_iota(jnp.int32, (QB, T), 1)
    causal = row >= col
    qs = q_ref[...]
    ks = k_ref[...]
    vs = v_ref[...]
    for h in range(NUM_HEADS):
        kh = h // REP
        q = qs[:, h * HEAD_DIM:(h + 1) * HEAD_DIM]
        k = ks[:, kh * HEAD_DIM:(kh + 1) * HEAD_DIM]
        v = vs[:, kh * HEAD_DIM:(kh + 1) * HEAD_DIM]
        s = lax.dot_general(q, k, (((1,), (1,)), ((), ())),
                            preferred_element_type=jnp.float32)
        s = jnp.where(causal, s, jnp.finfo(jnp.float32).min)
        m = jnp.max(s, axis=-1, keepdims=True)
        p = jnp.exp(s - m)
        l = jnp.sum(p, axis=-1, keepdims=True)
        o = lax.dot_general(p.astype(jnp.bfloat16), v, (((1,), (0,)), ((), ())),
                            preferred_element_type=jnp.float32)
        o_ref[:, h * HEAD_DIM:(h + 1) * HEAD_DIM] = o / l


def _oproj_body(a_ref, wo_ref, res_ref, wpost_ref, out_ref):
    o = lax.dot_general(a_ref[...], wo_ref[...], (((1,), (1,)), ((), ())),
                        preferred_element_type=jnp.float32)
    r = o + res_ref[...]
    var = jnp.mean(r * r, axis=-1, keepdims=True)
    out_ref[...] = r * lax.rsqrt(var + EPS) * wpost_ref[...]


def _topk_body(el_ref, tw_ref, ti_ref):
    x = el_ref[...]
    iota = lax.broadcasted_iota(jnp.int32, (T, NUM_EXPERTS), 1)
    vals = []
    ids = []
    for _ in range(TOP_K):
        m = jnp.max(x, axis=1, keepdims=True)
        eq = x == m
        idx = jnp.min(jnp.where(eq, iota, NUM_EXPERTS), axis=1, keepdims=True)
        vals.append(m)
        ids.append(idx)
        x = jnp.where(iota == idx, -1.0, x)
    tw = jnp.concatenate(vals, axis=1)
    ti = jnp.concatenate(ids, axis=1)
    tw_ref[...] = tw / jnp.sum(tw, axis=1, keepdims=True)
    ti_ref[...] = ti


RK = 128


def _rank_body(flat_ref, rank_ref):
    x = flat_ref[...]
    cpr = lax.broadcasted_iota(jnp.int32, (RK, RK), 0)
    cpc = lax.broadcasted_iota(jnp.int32, (RK, RK), 1)
    triu_incl = (cpr <= cpc).astype(jnp.bfloat16)
    tril_strict = (cpc < cpr).astype(jnp.bfloat16)
    occ_acc = jnp.zeros((RK, RK), dtype=jnp.float32)
    off_acc = jnp.zeros((RK, RK), dtype=jnp.float32)
    running = jnp.zeros((1, 1), dtype=jnp.float32)
    for e in range(NUM_EXPERTS):
        mask = (x == e)
        maskb = mask.astype(jnp.bfloat16)
        pref = lax.dot_general(maskb, triu_incl, (((1,), (0,)), ((), ())),
                               preferred_element_type=jnp.float32)
        rtot = pref[:, RK - 1:RK]
        rowoff = lax.dot_general(tril_strict, rtot.astype(jnp.bfloat16),
                                 (((1,), (0,)), ((), ())),
                                 preferred_element_type=jnp.float32)
        occ_e = pref + rowoff
        maskf = mask.astype(jnp.float32)
        occ_acc = occ_acc + maskf * occ_e
        off_acc = off_acc + maskf * running
        running = running + rowoff[RK - 1:RK, 0:1] + rtot[RK - 1:RK, 0:1]
    rank = occ_acc - 1.0 + off_acc
    rank_ref[...] = jnp.floor(rank + 0.5).astype(jnp.int32)


def _sc_scatter_body(rank_hbm, vals_hbm, out_hbm, idx_v, vals_v, sem):
    wid = lax.axis_index("s") * SC_NC + lax.axis_index("c")
    base = wid * ROWS_PER_W
    pltpu.sync_copy(rank_hbm.at[wid], idx_v)
    pltpu.sync_copy(vals_hbm.at[pl.ds(base, ROWS_PER_W)], vals_v)
    for j in range(ROWS_PER_W // 128):
        pltpu.async_copy(vals_v.at[pl.ds(j * 128, 128)],
                         out_hbm.at[idx_v.at[j]], sem).wait()


def _invert_rank_sc(rank3, vals):
    mesh = plsc.VectorSubcoreMesh(core_axis_name="c", subcore_axis_name="s")
    kfn = pl.kernel(
        _sc_scatter_body,
        out_type=jax.ShapeDtypeStruct((N_PERM, 128), jnp.int32),
        mesh=mesh,
        scratch_types=[
            pltpu.VMEM((ROWS_PER_W // 128, 128), jnp.int32),
            pltpu.VMEM((ROWS_PER_W, 128), jnp.int32),
            pltpu.SemaphoreType.DMA,
        ],
    )
    return kfn(rank3, vals)


def _sc_gather_body(table_hbm, idx_hbm, out_hbm, idx_v, buf0, buf1, sem0, sem1):
    wid = lax.axis_index("s") * SC_NC + lax.axis_index("c")
    base = wid * ROWS_PER_W
    pltpu.sync_copy(idx_hbm.at[pl.ds(base, ROWS_PER_W)], idx_v)
    nchunk = ROWS_PER_W // GCHUNK
    bufs = (buf0, buf1)
    sems = (sem0, sem1)
    handles = [None, None]
    handles[0] = pltpu.async_copy(
        table_hbm.at[idx_v.at[pl.ds(0, GCHUNK)]], bufs[0], sems[0])
    for c in range(1, nchunk + 1):
        if c < nchunk:
            b = c % 2
            handles[b] = pltpu.async_copy(
                table_hbm.at[idx_v.at[pl.ds(c * GCHUNK, GCHUNK)]], bufs[b], sems[b])
        pb = (c - 1) % 2
        handles[pb].wait()
        pltpu.sync_copy(bufs[pb], out_hbm.at[pl.ds(base + (c - 1) * GCHUNK, GCHUNK)])


def _permute_gather_sc(out, src_idx):
    mesh = plsc.VectorSubcoreMesh(core_axis_name="c", subcore_axis_name="s")
    kfn = pl.kernel(
        _sc_gather_body,
        out_type=jax.ShapeDtypeStruct((N_PERM, HIDDEN), jnp.float32),
        mesh=mesh,
        scratch_types=[
            pltpu.VMEM((ROWS_PER_W,), jnp.int32),
            pltpu.VMEM((GCHUNK, HIDDEN), jnp.float32),
            pltpu.VMEM((GCHUNK, HIDDEN), jnp.float32),
            pltpu.SemaphoreType.DMA,
            pltpu.SemaphoreType.DMA,
        ],
    )
    return kfn(out, src_idx)


def kernel(positions, hidden_states, kv_cache, w_pre, W_qkv, W_o, w_post, W_gate):
    pos2d = positions.astype(jnp.float32).reshape(T, 1)
    wpre2d = w_pre.reshape(1, HIDDEN)
    wpost2d = w_post.reshape(1, HIDDEN)

    q, k, v = pl.pallas_call(
        _qkv_body,
        grid=(T // TB,),
        in_specs=[
            pl.BlockSpec((TB, 1), lambda i: (i, 0)),
            pl.BlockSpec((TB, HIDDEN), lambda i: (i, 0)),
            pl.BlockSpec((1, HIDDEN), lambda i: (0, 0)),
            pl.BlockSpec((Q_SIZE + 2 * KV_SIZE, HIDDEN), lambda i: (0, 0)),
        ],
        out_specs=[
            pl.BlockSpec((TB, Q_SIZE), lambda i: (i, 0)),
            pl.BlockSpec((TB, KV_SIZE), lambda i: (i, 0)),
            pl.BlockSpec((TB, KV_SIZE), lambda i: (i, 0)),
        ],
        out_shape=[
            jax.ShapeDtypeStruct((T, Q_SIZE), jnp.bfloat16),
            jax.ShapeDtypeStruct((T, KV_SIZE), jnp.bfloat16),
            jax.ShapeDtypeStruct((T, KV_SIZE), jnp.bfloat16),
        ],
    )(pos2d, hidden_states, wpre2d, W_qkv)

    attn = pl.pallas_call(
        _attn_body,
        grid=(T // QB,),
        in_specs=[
            pl.BlockSpec((QB, Q_SIZE), lambda qb: (qb, 0)),
            pl.BlockSpec((T, KV_SIZE), lambda qb: (0, 0)),
            pl.BlockSpec((T, KV_SIZE), lambda qb: (0, 0)),
        ],
        out_specs=pl.BlockSpec((QB, Q_SIZE), lambda qb: (qb, 0)),
        out_shape=jax.ShapeDtypeStruct((T, Q_SIZE), jnp.float32),
    )(q, k, v)

    out = pl.pallas_call(
        _oproj_body,
        grid=(T // TB,),
        in_specs=[
            pl.BlockSpec((TB, Q_SIZE), lambda i: (i, 0)),
            pl.BlockSpec((HIDDEN, Q_SIZE), lambda i: (0, 0)),
            pl.BlockSpec((TB, HIDDEN), lambda i: (i, 0)),
            pl.BlockSpec((1, HIDDEN), lambda i: (0, 0)),
        ],
        out_specs=pl.BlockSpec((TB, HIDDEN), lambda i: (i, 0)),
        out_shape=jax.ShapeDtypeStruct((T, HIDDEN), jnp.float32),
    )(attn, W_o, hidden_states, wpost2d)

    key_r = jax.random.fold_in(jax.random.key(0), 123)
    expert_logits = jax.random.uniform(key_r, (T, NUM_EXPERTS), dtype=jnp.float32)
    topk_weights, topk_ids = pl.pallas_call(
        _topk_body,
        out_shape=[
            jax.ShapeDtypeStruct((T, TOP_K), jnp.float32),
            jax.ShapeDtypeStruct((T, TOP_K), jnp.int32),
        ],
    )(expert_logits)
    flat2 = topk_ids.reshape(RK, RK)
    rank2 = pl.pallas_call(
        _rank_body,
        out_shape=jax.ShapeDtypeStruct((RK, RK), jnp.int32),
    )(flat2)
    rank3 = rank2.reshape(NW, ROWS_PER_W // 128, 128)
    ii = jnp.arange(N_PERM, dtype=jnp.int32)
    vals = jnp.stack([ii, ii // TOP_K], axis=1)
    vals = jnp.pad(vals, ((0, 0), (0, 126)))
    inv = _invert_rank_sc(rank3, vals)
    reorder_ids = inv[:, 0]
    src_idx = inv[:, 1]

    permuted_output = _permute_gather_sc(out, src_idx)
    return (permuted_output, topk_weights, topk_ids, reorder_ids)


# speedup vs baseline: 1.2026x; 1.0373x over previous
import numpy as np
import jax
import jax.numpy as jnp
from jax import lax
from jax.experimental import pallas as pl
from jax.experimental.pallas import tpu as pltpu
from jax.experimental.pallas import tpu_sc as plsc

HIDDEN = 768
NUM_HEADS = 12
NUM_KV_HEADS = 4
HEAD_DIM = 64
HALF = HEAD_DIM // 2
NUM_EXPERTS = 64
TOP_K = 8
ROPE_THETA = 10000.0
T = 2048
Q_SIZE = NUM_HEADS * HEAD_DIM
KV_SIZE = NUM_KV_HEADS * HEAD_DIM
SCALING = HEAD_DIM ** -0.5
EPS = 1e-6
REP = NUM_HEADS // NUM_KV_HEADS

TB = 256
QB = 512

SC_NC = 2
SC_NS = 16
NW = SC_NC * SC_NS
N_PERM = T * TOP_K
ROWS_PER_W = N_PERM // NW
GCHUNK = 64


def _qkv_body(pos_ref, x_ref, wpre_ref, wqkv_ref, q_ref, k_ref, v_ref):
    x = x_ref[...]
    var = jnp.mean(x * x, axis=-1, keepdims=True)
    h = (x * lax.rsqrt(var + EPS) * wpre_ref[...]).astype(jnp.bfloat16)
    qkv = lax.dot_general(h, wqkv_ref[...], (((1,), (1,)), ((), ())),
                          preferred_element_type=jnp.float32)
    pos = pos_ref[...]
    expo = lax.broadcasted_iota(jnp.int32, (1, HALF), 1).astype(jnp.float32) * (2.0 / HEAD_DIM)
    inv_freq = jnp.exp(expo * (-float(np.log(ROPE_THETA))))
    freqs = pos * inv_freq
    c = jnp.cos(freqs)
    s = jnp.sin(freqs)

    def rot(x):
        x1 = x[:, :HALF]
        x2 = x[:, HALF:]
        return jnp.concatenate([x1 * c - x2 * s, x2 * c + x1 * s], axis=1)

    for hh in range(NUM_HEADS):
        sl = slice(hh * HEAD_DIM, (hh + 1) * HEAD_DIM)
        q_ref[:, sl] = (rot(qkv[:, sl]) * SCALING).astype(jnp.bfloat16)
    for kh in range(NUM_KV_HEADS):
        sl = slice(Q_SIZE + kh * HEAD_DIM, Q_SIZE + (kh + 1) * HEAD_DIM)
        k_ref[:, kh * HEAD_DIM:(kh + 1) * HEAD_DIM] = rot(qkv[:, sl]).astype(jnp.bfloat16)
    v_ref[...] = qkv[:, Q_SIZE + KV_SIZE:].astype(jnp.bfloat16)


def _attn_body(q_ref, k_ref, v_ref, o_ref):
    qb = pl.program_id(0)
    row = qb * QB + lax.broadcasted_iota(jnp.int32, (QB, T), 0)
    col = lax.broadcasted_iota(jnp.int32, (QB, T), 1)
    causal = row >= col
    qs = q_ref[...]
    ks = k_ref[...]
    vs = v_ref[...]
    for h in range(NUM_HEADS):
        kh = h // REP
        q = qs[:, h * HEAD_DIM:(h + 1) * HEAD_DIM]
        k = ks[:, kh * HEAD_DIM:(kh + 1) * HEAD_DIM]
        v = vs[:, kh * HEAD_DIM:(kh + 1) * HEAD_DIM]
        s = lax.dot_general(q, k, (((1,), (1,)), ((), ())),
                            preferred_element_type=jnp.float32)
        s = jnp.where(causal, s, jnp.finfo(jnp.float32).min)
        m = jnp.max(s, axis=-1, keepdims=True)
        p = jnp.exp(s - m)
        l = jnp.sum(p, axis=-1, keepdims=True)
        o = lax.dot_general(p.astype(jnp.bfloat16), v, (((1,), (0,)), ((), ())),
                            preferred_element_type=jnp.float32)
        o_ref[:, h * HEAD_DIM:(h + 1) * HEAD_DIM] = o / l


def _oproj_body(a_ref, wo_ref, res_ref, wpost_ref, out_ref):
    o = lax.dot_general(a_ref[...].astype(jnp.bfloat16), wo_ref[...],
                        (((1,), (1,)), ((), ())),
                        preferred_element_type=jnp.float32)
    r = o + res_ref[...]
    var = jnp.mean(r * r, axis=-1, keepdims=True)
    out_ref[...] = r * lax.rsqrt(var + EPS) * wpost_ref[...]


def _topk_body(el_ref, tw_ref, ti_ref):
    x = el_ref[...]
    iota = lax.broadcasted_iota(jnp.int32, (T, NUM_EXPERTS), 1)
    vals = []
    ids = []
    for _ in range(TOP_K):
        m = jnp.max(x, axis=1, keepdims=True)
        eq = x == m
        idx = jnp.min(jnp.where(eq, iota, NUM_EXPERTS), axis=1, keepdims=True)
        vals.append(m)
        ids.append(idx)
        x = jnp.where(iota == idx, -1.0, x)
    tw = jnp.concatenate(vals, axis=1)
    ti = jnp.concatenate(ids, axis=1)
    tw_ref[...] = tw / jnp.sum(tw, axis=1, keepdims=True)
    ti_ref[...] = ti


RK = 128


def _rank_body(flat_ref, rank_ref):
    x = flat_ref[...]
    cpr = lax.broadcasted_iota(jnp.int32, (RK, RK), 0)
    cpc = lax.broadcasted_iota(jnp.int32, (RK, RK), 1)
    triu_incl = (cpr <= cpc).astype(jnp.bfloat16)
    tril_strict = (cpc < cpr).astype(jnp.bfloat16)
    occ_acc = jnp.zeros((RK, RK), dtype=jnp.float32)
    off_acc = jnp.zeros((RK, RK), dtype=jnp.float32)
    running = jnp.zeros((1, 1), dtype=jnp.float32)
    for e in range(NUM_EXPERTS):
        mask = (x == e)
        maskb = mask.astype(jnp.bfloat16)
        pref = lax.dot_general(maskb, triu_incl, (((1,), (0,)), ((), ())),
                               preferred_element_type=jnp.float32)
        rtot = pref[:, RK - 1:RK]
        rowoff = lax.dot_general(tril_strict, rtot.astype(jnp.bfloat16),
                                 (((1,), (0,)), ((), ())),
                                 preferred_element_type=jnp.float32)
        occ_e = pref + rowoff
        maskf = mask.astype(jnp.float32)
        occ_acc = occ_acc + maskf * occ_e
        off_acc = off_acc + maskf * running
        running = running + rowoff[RK - 1:RK, 0:1] + rtot[RK - 1:RK, 0:1]
    rank = occ_acc - 1.0 + off_acc
    rank_ref[...] = jnp.floor(rank + 0.5).astype(jnp.int32)


def _sc_scatter_body(rank_hbm, vals_hbm, out_hbm, idx_v, vals_v, sem):
    wid = lax.axis_index("s") * SC_NC + lax.axis_index("c")
    base = wid * ROWS_PER_W
    pltpu.sync_copy(rank_hbm.at[wid], idx_v)
    pltpu.sync_copy(vals_hbm.at[pl.ds(base, ROWS_PER_W)], vals_v)
    for j in range(ROWS_PER_W // 128):
        pltpu.async_copy(vals_v.at[pl.ds(j * 128, 128)],
                         out_hbm.at[idx_v.at[j]], sem).wait()


def _invert_rank_sc(rank3, vals):
    mesh = plsc.VectorSubcoreMesh(core_axis_name="c", subcore_axis_name="s")
    kfn = pl.kernel(
        _sc_scatter_body,
        out_type=jax.ShapeDtypeStruct((N_PERM, 128), jnp.int32),
        mesh=mesh,
        scratch_types=[
            pltpu.VMEM((ROWS_PER_W // 128, 128), jnp.int32),
            pltpu.VMEM((ROWS_PER_W, 128), jnp.int32),
            pltpu.SemaphoreType.DMA,
        ],
    )
    return kfn(rank3, vals)


def _sc_gather_body(table_hbm, idx_hbm, out_hbm, idx_v, buf0, buf1, sem0, sem1):
    wid = lax.axis_index("s") * SC_NC + lax.axis_index("c")
    base = wid * ROWS_PER_W
    pltpu.sync_copy(idx_hbm.at[pl.ds(base, ROWS_PER_W)], idx_v)
    nchunk = ROWS_PER_W // GCHUNK
    bufs = (buf0, buf1)
    sems = (sem0, sem1)
    handles = [None, None]
    handles[0] = pltpu.async_copy(
        table_hbm.at[idx_v.at[pl.ds(0, GCHUNK)]], bufs[0], sems[0])
    for c in range(1, nchunk + 1):
        if c < nchunk:
            b = c % 2
            handles[b] = pltpu.async_copy(
                table_hbm.at[idx_v.at[pl.ds(c * GCHUNK, GCHUNK)]], bufs[b], sems[b])
        pb = (c - 1) % 2
        handles[pb].wait()
        pltpu.sync_copy(bufs[pb], out_hbm.at[pl.ds(base + (c - 1) * GCHUNK, GCHUNK)])


def _permute_gather_sc(out, src_idx):
    mesh = plsc.VectorSubcoreMesh(core_axis_name="c", subcore_axis_name="s")
    kfn = pl.kernel(
        _sc_gather_body,
        out_type=jax.ShapeDtypeStruct((N_PERM, HIDDEN), jnp.float32),
        mesh=mesh,
        scratch_types=[
            pltpu.VMEM((ROWS_PER_W,), jnp.int32),
            pltpu.VMEM((GCHUNK, HIDDEN), jnp.float32),
            pltpu.VMEM((GCHUNK, HIDDEN), jnp.float32),
            pltpu.SemaphoreType.DMA,
            pltpu.SemaphoreType.DMA,
        ],
    )
    return kfn(out, src_idx)


def kernel(positions, hidden_states, kv_cache, w_pre, W_qkv, W_o, w_post, W_gate):
    pos2d = positions.astype(jnp.float32).reshape(T, 1)
    wpre2d = w_pre.reshape(1, HIDDEN)
    wpost2d = w_post.reshape(1, HIDDEN)

    q, k, v = pl.pallas_call(
        _qkv_body,
        grid=(T // TB,),
        in_specs=[
            pl.BlockSpec((TB, 1), lambda i: (i, 0)),
            pl.BlockSpec((TB, HIDDEN), lambda i: (i, 0)),
            pl.BlockSpec((1, HIDDEN), lambda i: (0, 0)),
            pl.BlockSpec((Q_SIZE + 2 * KV_SIZE, HIDDEN), lambda i: (0, 0)),
        ],
        out_specs=[
            pl.BlockSpec((TB, Q_SIZE), lambda i: (i, 0)),
            pl.BlockSpec((TB, KV_SIZE), lambda i: (i, 0)),
            pl.BlockSpec((TB, KV_SIZE), lambda i: (i, 0)),
        ],
        out_shape=[
            jax.ShapeDtypeStruct((T, Q_SIZE), jnp.bfloat16),
            jax.ShapeDtypeStruct((T, KV_SIZE), jnp.bfloat16),
            jax.ShapeDtypeStruct((T, KV_SIZE), jnp.bfloat16),
        ],
    )(pos2d, hidden_states, wpre2d, W_qkv.astype(jnp.bfloat16))

    attn = pl.pallas_call(
        _attn_body,
        grid=(T // QB,),
        in_specs=[
            pl.BlockSpec((QB, Q_SIZE), lambda qb: (qb, 0)),
            pl.BlockSpec((T, KV_SIZE), lambda qb: (0, 0)),
            pl.BlockSpec((T, KV_SIZE), lambda qb: (0, 0)),
        ],
        out_specs=pl.BlockSpec((QB, Q_SIZE), lambda qb: (qb, 0)),
        out_shape=jax.ShapeDtypeStruct((T, Q_SIZE), jnp.float32),
    )(q, k, v)

    out = pl.pallas_call(
        _oproj_body,
        grid=(T // TB,),
        in_specs=[
            pl.BlockSpec((TB, Q_SIZE), lambda i: (i, 0)),
            pl.BlockSpec((HIDDEN, Q_SIZE), lambda i: (0, 0)),
            pl.BlockSpec((TB, HIDDEN), lambda i: (i, 0)),
            pl.BlockSpec((1, HIDDEN), lambda i: (0, 0)),
        ],
        out_specs=pl.BlockSpec((TB, HIDDEN), lambda i: (i, 0)),
        out_shape=jax.ShapeDtypeStruct((T, HIDDEN), jnp.float32),
    )(attn, W_o.astype(jnp.bfloat16), hidden_states, wpost2d)

    key_r = jax.random.fold_in(jax.random.key(0), 123)
    expert_logits = jax.random.uniform(key_r, (T, NUM_EXPERTS), dtype=jnp.float32)
    topk_weights, topk_ids = pl.pallas_call(
        _topk_body,
        out_shape=[
            jax.ShapeDtypeStruct((T, TOP_K), jnp.float32),
            jax.ShapeDtypeStruct((T, TOP_K), jnp.int32),
        ],
    )(expert_logits)
    flat2 = topk_ids.reshape(RK, RK)
    rank2 = pl.pallas_call(
        _rank_body,
        out_shape=jax.ShapeDtypeStruct((RK, RK), jnp.int32),
    )(flat2)
    rank3 = rank2.reshape(NW, ROWS_PER_W // 128, 128)
    ii = jnp.arange(N_PERM, dtype=jnp.int32)
    vals = jnp.pad(ii[:, None], ((0, 0), (0, 127)))
    inv = _invert_rank_sc(rank3, vals)
    reorder_ids = inv[:, 0]
    src_idx = lax.shift_right_logical(reorder_ids, 3)

    permuted_output = _permute_gather_sc(out, src_idx)
    return (permuted_output, topk_weights, topk_ids, reorder_ids)


# speedup vs baseline: 1.3181x; 1.0961x over previous
import numpy as np
import jax
import jax.numpy as jnp
from jax import lax
from jax.experimental import pallas as pl
from jax.experimental.pallas import tpu as pltpu
from jax.experimental.pallas import tpu_sc as plsc

HIDDEN = 768
NUM_HEADS = 12
NUM_KV_HEADS = 4
HEAD_DIM = 64
HALF = HEAD_DIM // 2
NUM_EXPERTS = 64
TOP_K = 8
ROPE_THETA = 10000.0
T = 2048
Q_SIZE = NUM_HEADS * HEAD_DIM
KV_SIZE = NUM_KV_HEADS * HEAD_DIM
SCALING = HEAD_DIM ** -0.5
EPS = 1e-6
REP = NUM_HEADS // NUM_KV_HEADS

TB = 256
QB = 512

SC_NC = 2
SC_NS = 16
NW = SC_NC * SC_NS
N_PERM = T * TOP_K
ROWS_PER_W = N_PERM // NW
GCHUNK = 64


def _qkv_body(pos_ref, x_ref, wpre_ref, wqkv_ref, q_ref, k_ref, v_ref):
    x = x_ref[...]
    var = jnp.mean(x * x, axis=-1, keepdims=True)
    h = (x * lax.rsqrt(var + EPS) * wpre_ref[...]).astype(jnp.bfloat16)
    qkv = lax.dot_general(h, wqkv_ref[...], (((1,), (1,)), ((), ())),
                          preferred_element_type=jnp.float32)
    pos = pos_ref[...]
    expo = lax.broadcasted_iota(jnp.int32, (1, HALF), 1).astype(jnp.float32) * (2.0 / HEAD_DIM)
    inv_freq = jnp.exp(expo * (-float(np.log(ROPE_THETA))))
    freqs = pos * inv_freq
    c = jnp.cos(freqs)
    s = jnp.sin(freqs)

    def rot(x):
        x1 = x[:, :HALF]
        x2 = x[:, HALF:]
        return jnp.concatenate([x1 * c - x2 * s, x2 * c + x1 * s], axis=1)

    for hh in range(NUM_HEADS):
        sl = slice(hh * HEAD_DIM, (hh + 1) * HEAD_DIM)
        q_ref[:, sl] = (rot(qkv[:, sl]) * SCALING).astype(jnp.bfloat16)
    for kh in range(NUM_KV_HEADS):
        sl = slice(Q_SIZE + kh * HEAD_DIM, Q_SIZE + (kh + 1) * HEAD_DIM)
        k_ref[:, kh * HEAD_DIM:(kh + 1) * HEAD_DIM] = rot(qkv[:, sl]).astype(jnp.bfloat16)
    v_ref[...] = qkv[:, Q_SIZE + KV_SIZE:].astype(jnp.bfloat16)


def _attn_body(q_ref, k_ref, v_ref, o_ref):
    qb = pl.program_id(0)
    row = qb * QB + lax.broadcasted_iota(jnp.int32, (QB, T), 0)
    col = lax.broadcasted_iota(jnp.int32, (QB, T), 1)
    causal = row >= col
    qs = q_ref[...]
    ks = k_ref[...]
    vs = v_ref[...]
    for h in range(NUM_HEADS):
        kh = h // REP
        q = qs[:, h * HEAD_DIM:(h + 1) * HEAD_DIM]
        k = ks[:, kh * HEAD_DIM:(kh + 1) * HEAD_DIM]
        v = vs[:, kh * HEAD_DIM:(kh + 1) * HEAD_DIM]
        s = lax.dot_general(q, k, (((1,), (1,)), ((), ())),
                            preferred_element_type=jnp.float32)
        s = jnp.where(causal, s, jnp.finfo(jnp.float32).min)
        m = jnp.max(s, axis=-1, keepdims=True)
        p = jnp.exp(s - m)
        l = jnp.sum(p, axis=-1, keepdims=True)
        o = lax.dot_general(p.astype(jnp.bfloat16), v, (((1,), (0,)), ((), ())),
                            preferred_element_type=jnp.float32)
        o_ref[:, h * HEAD_DIM:(h + 1) * HEAD_DIM] = o / l


def _oproj_body(a_ref, wo_ref, res_ref, wpost_ref, out_ref):
    o = lax.dot_general(a_ref[...].astype(jnp.bfloat16), wo_ref[...],
                        (((1,), (1,)), ((), ())),
                        preferred_element_type=jnp.float32)
    r = o + res_ref[...]
    var = jnp.mean(r * r, axis=-1, keepdims=True)
    out_ref[...] = r * lax.rsqrt(var + EPS) * wpost_ref[...]


def _topk_body(el_ref, tw_ref, ti_ref):
    x = el_ref[...]
    iota = lax.broadcasted_iota(jnp.int32, (T, NUM_EXPERTS), 1)
    vals = []
    ids = []
    for _ in range(TOP_K):
        m = jnp.max(x, axis=1, keepdims=True)
        eq = x == m
        idx = jnp.min(jnp.where(eq, iota, NUM_EXPERTS), axis=1, keepdims=True)
        vals.append(m)
        ids.append(idx)
        x = jnp.where(iota == idx, -1.0, x)
    tw = jnp.concatenate(vals, axis=1)
    ti = jnp.concatenate(ids, axis=1)
    tw_ref[...] = tw / jnp.sum(tw, axis=1, keepdims=True)
    ti_ref[...] = ti


RK = 128


def _rank_body(flat_ref, rank_ref):
    x = flat_ref[...]
    cpr = lax.broadcasted_iota(jnp.int32, (RK, RK), 0)
    cpc = lax.broadcasted_iota(jnp.int32, (RK, RK), 1)
    triu_incl = (cpr <= cpc).astype(jnp.bfloat16)
    tril_strict = (cpc < cpr).astype(jnp.bfloat16)
    occ_acc = jnp.zeros((RK, RK), dtype=jnp.float32)
    off_acc = jnp.zeros((RK, RK), dtype=jnp.float32)
    running = jnp.zeros((1, 1), dtype=jnp.float32)
    for e in range(NUM_EXPERTS):
        mask = (x == e)
        maskb = mask.astype(jnp.bfloat16)
        pref = lax.dot_general(maskb, triu_incl, (((1,), (0,)), ((), ())),
                               preferred_element_type=jnp.float32)
        rtot = pref[:, RK - 1:RK]
        rowoff = lax.dot_general(tril_strict, rtot.astype(jnp.bfloat16),
                                 (((1,), (0,)), ((), ())),
                                 preferred_element_type=jnp.float32)
        occ_e = pref + rowoff
        maskf = mask.astype(jnp.float32)
        occ_acc = occ_acc + maskf * occ_e
        off_acc = off_acc + maskf * running
        running = running + rowoff[RK - 1:RK, 0:1] + rtot[RK - 1:RK, 0:1]
    rank = occ_acc - 1.0 + off_acc
    rank_ref[...] = jnp.floor(rank + 0.5).astype(jnp.int32)


def _sc_scatter_body(rank_hbm, vals_hbm, out_hbm, idx_v, vals_v, sem):
    wid = lax.axis_index("s") * SC_NC + lax.axis_index("c")
    base = wid * ROWS_PER_W
    pltpu.sync_copy(rank_hbm.at[wid], idx_v)
    pltpu.sync_copy(vals_hbm.at[pl.ds(base, ROWS_PER_W)], vals_v)
    for j in range(ROWS_PER_W // 128):
        pltpu.async_copy(vals_v.at[pl.ds(j * 128, 128)],
                         out_hbm.at[idx_v.at[j]], sem).wait()


def _invert_rank_sc(rank3, vals):
    mesh = plsc.VectorSubcoreMesh(core_axis_name="c", subcore_axis_name="s")
    kfn = pl.kernel(
        _sc_scatter_body,
        out_type=jax.ShapeDtypeStruct((N_PERM, 128), jnp.int32),
        mesh=mesh,
        scratch_types=[
            pltpu.VMEM((ROWS_PER_W // 128, 128), jnp.int32),
            pltpu.VMEM((ROWS_PER_W, 128), jnp.int32),
            pltpu.SemaphoreType.DMA,
        ],
    )
    return kfn(rank3, vals)


def _sc_permute_body(out_hbm, rankt_hbm, perm_hbm, stage_v, idx_v, sem):
    wid = lax.axis_index("s") * SC_NC + lax.axis_index("c")
    pltpu.sync_copy(out_hbm.at[pl.ds(wid * (T // NW), T // NW)], stage_v)
    pltpu.sync_copy(rankt_hbm.at[wid], idx_v)
    handles = [pltpu.async_copy(stage_v, perm_hbm.at[idx_v.at[j]], sem)
               for j in range(TOP_K)]
    for h in handles:
        h.wait()


def _permute_scatter_sc(out, rankt3):
    mesh = plsc.VectorSubcoreMesh(core_axis_name="c", subcore_axis_name="s")
    kfn = pl.kernel(
        _sc_permute_body,
        out_type=jax.ShapeDtypeStruct((N_PERM, HIDDEN), jnp.float32),
        mesh=mesh,
        scratch_types=[
            pltpu.VMEM((T // NW, HIDDEN), jnp.float32),
            pltpu.VMEM((TOP_K, T // NW), jnp.int32),
            pltpu.SemaphoreType.DMA,
        ],
    )
    return kfn(out, rankt3)


def kernel(positions, hidden_states, kv_cache, w_pre, W_qkv, W_o, w_post, W_gate):
    pos2d = positions.astype(jnp.float32).reshape(T, 1)
    wpre2d = w_pre.reshape(1, HIDDEN)
    wpost2d = w_post.reshape(1, HIDDEN)

    q, k, v = pl.pallas_call(
        _qkv_body,
        grid=(T // TB,),
        in_specs=[
            pl.BlockSpec((TB, 1), lambda i: (i, 0)),
            pl.BlockSpec((TB, HIDDEN), lambda i: (i, 0)),
            pl.BlockSpec((1, HIDDEN), lambda i: (0, 0)),
            pl.BlockSpec((Q_SIZE + 2 * KV_SIZE, HIDDEN), lambda i: (0, 0)),
        ],
        out_specs=[
            pl.BlockSpec((TB, Q_SIZE), lambda i: (i, 0)),
            pl.BlockSpec((TB, KV_SIZE), lambda i: (i, 0)),
            pl.BlockSpec((TB, KV_SIZE), lambda i: (i, 0)),
        ],
        out_shape=[
            jax.ShapeDtypeStruct((T, Q_SIZE), jnp.bfloat16),
            jax.ShapeDtypeStruct((T, KV_SIZE), jnp.bfloat16),
            jax.ShapeDtypeStruct((T, KV_SIZE), jnp.bfloat16),
        ],
    )(pos2d, hidden_states, wpre2d, W_qkv.astype(jnp.bfloat16))

    attn = pl.pallas_call(
        _attn_body,
        grid=(T // QB,),
        in_specs=[
            pl.BlockSpec((QB, Q_SIZE), lambda qb: (qb, 0)),
            pl.BlockSpec((T, KV_SIZE), lambda qb: (0, 0)),
            pl.BlockSpec((T, KV_SIZE), lambda qb: (0, 0)),
        ],
        out_specs=pl.BlockSpec((QB, Q_SIZE), lambda qb: (qb, 0)),
        out_shape=jax.ShapeDtypeStruct((T, Q_SIZE), jnp.float32),
    )(q, k, v)

    out = pl.pallas_call(
        _oproj_body,
        grid=(T // TB,),
        in_specs=[
            pl.BlockSpec((TB, Q_SIZE), lambda i: (i, 0)),
            pl.BlockSpec((HIDDEN, Q_SIZE), lambda i: (0, 0)),
            pl.BlockSpec((TB, HIDDEN), lambda i: (i, 0)),
            pl.BlockSpec((1, HIDDEN), lambda i: (0, 0)),
        ],
        out_specs=pl.BlockSpec((TB, HIDDEN), lambda i: (i, 0)),
        out_shape=jax.ShapeDtypeStruct((T, HIDDEN), jnp.float32),
    )(attn, W_o.astype(jnp.bfloat16), hidden_states, wpost2d)

    key_r = jax.random.fold_in(jax.random.key(0), 123)
    expert_logits = jax.random.uniform(key_r, (T, NUM_EXPERTS), dtype=jnp.float32)
    topk_weights, topk_ids = pl.pallas_call(
        _topk_body,
        out_shape=[
            jax.ShapeDtypeStruct((T, TOP_K), jnp.float32),
            jax.ShapeDtypeStruct((T, TOP_K), jnp.int32),
        ],
    )(expert_logits)
    flat2 = topk_ids.reshape(RK, RK)
    rank2 = pl.pallas_call(
        _rank_body,
        out_shape=jax.ShapeDtypeStruct((RK, RK), jnp.int32),
    )(flat2)
    rank3 = rank2.reshape(NW, ROWS_PER_W // 128, 128)
    ii = jnp.arange(N_PERM, dtype=jnp.int32)
    vals = jnp.pad(ii[:, None], ((0, 0), (0, 127)))
    inv = _invert_rank_sc(rank3, vals)
    reorder_ids = inv[:, 0]
    rankt3 = rank2.reshape(NW, T // NW, TOP_K).transpose(0, 2, 1)
    permuted_output = _permute_scatter_sc(out, rankt3)
    return (permuted_output, topk_weights, topk_ids, reorder_ids)


# speedup vs baseline: 1.3189x; 1.0006x over previous
import numpy as np
import jax
import jax.numpy as jnp
from jax import lax
from jax.experimental import pallas as pl
from jax.experimental.pallas import tpu as pltpu
from jax.experimental.pallas import tpu_sc as plsc

HIDDEN = 768
NUM_HEADS = 12
NUM_KV_HEADS = 4
HEAD_DIM = 64
HALF = HEAD_DIM // 2
NUM_EXPERTS = 64
TOP_K = 8
ROPE_THETA = 10000.0
T = 2048
Q_SIZE = NUM_HEADS * HEAD_DIM
KV_SIZE = NUM_KV_HEADS * HEAD_DIM
SCALING = HEAD_DIM ** -0.5
EPS = 1e-6
REP = NUM_HEADS // NUM_KV_HEADS

TB = 256
QB = 512

SC_NC = 2
SC_NS = 16
NW = SC_NC * SC_NS
N_PERM = T * TOP_K
ROWS_PER_W = N_PERM // NW


def _qkv_body(pos_ref, x_ref, wpre_ref, wqkv_ref, q_ref, k_ref, v_ref):
    x = x_ref[...]
    var = jnp.mean(x * x, axis=-1, keepdims=True)
    h = (x * lax.rsqrt(var + EPS) * wpre_ref[...]).astype(jnp.bfloat16)
    qkv = lax.dot_general(h, wqkv_ref[...], (((1,), (1,)), ((), ())),
                          preferred_element_type=jnp.float32)
    pos = pos_ref[...]
    expo = lax.broadcasted_iota(jnp.int32, (1, HALF), 1).astype(jnp.float32) * (2.0 / HEAD_DIM)
    inv_freq = jnp.exp(expo * (-float(np.log(ROPE_THETA))))
    freqs = pos * inv_freq
    c = jnp.cos(freqs)
    s = jnp.sin(freqs)

    def rot(x):
        x1 = x[:, :HALF]
        x2 = x[:, HALF:]
        return jnp.concatenate([x1 * c - x2 * s, x2 * c + x1 * s], axis=1)

    for hh in range(NUM_HEADS):
        sl = slice(hh * HEAD_DIM, (hh + 1) * HEAD_DIM)
        q_ref[:, sl] = (rot(qkv[:, sl]) * SCALING).astype(jnp.bfloat16)
    for kh in range(NUM_KV_HEADS):
        sl = slice(Q_SIZE + kh * HEAD_DIM, Q_SIZE + (kh + 1) * HEAD_DIM)
        k_ref[:, kh * HEAD_DIM:(kh + 1) * HEAD_DIM] = rot(qkv[:, sl]).astype(jnp.bfloat16)
    v_ref[...] = qkv[:, Q_SIZE + KV_SIZE:].astype(jnp.bfloat16)


def _attn_body(q_ref, k_ref, v_ref, o_ref):
    qb = pl.program_id(0)
    row = qb * QB + lax.broadcasted_iota(jnp.int32, (QB, T), 0)
    col = lax.broadcasted_iota(jnp.int32, (QB, T), 1)
    causal = row >= col
    qs = q_ref[...]
    ks = k_ref[...]
    vs = v_ref[...]
    for h in range(NUM_HEADS):
        kh = h // REP
        q = qs[:, h * HEAD_DIM:(h + 1) * HEAD_DIM]
        k = ks[:, kh * HEAD_DIM:(kh + 1) * HEAD_DIM]
        v = vs[:, kh * HEAD_DIM:(kh + 1) * HEAD_DIM]
        s = lax.dot_general(q, k, (((1,), (1,)), ((), ())),
                            preferred_element_type=jnp.float32)
        s = jnp.where(causal, s, jnp.finfo(jnp.float32).min)
        m = jnp.max(s, axis=-1, keepdims=True)
        p = jnp.exp(s - m)
        l = jnp.sum(p, axis=-1, keepdims=True)
        o = lax.dot_general(p.astype(jnp.bfloat16), v, (((1,), (0,)), ((), ())),
                            preferred_element_type=jnp.float32)
        o_ref[:, h * HEAD_DIM:(h + 1) * HEAD_DIM] = o / l


def _oproj_body(a_ref, wo_ref, res_ref, wpost_ref, out_ref):
    o = lax.dot_general(a_ref[...].astype(jnp.bfloat16), wo_ref[...],
                        (((1,), (1,)), ((), ())),
                        preferred_element_type=jnp.float32)
    r = o + res_ref[...]
    var = jnp.mean(r * r, axis=-1, keepdims=True)
    out_ref[...] = r * lax.rsqrt(var + EPS) * wpost_ref[...]


def _topk_body(el_ref, tw_ref, ti_ref):
    x = el_ref[...]
    iota = lax.broadcasted_iota(jnp.int32, (T, NUM_EXPERTS), 1)
    vals = []
    ids = []
    for _ in range(TOP_K):
        m = jnp.max(x, axis=1, keepdims=True)
        eq = x == m
        idx = jnp.min(jnp.where(eq, iota, NUM_EXPERTS), axis=1, keepdims=True)
        vals.append(m)
        ids.append(idx)
        x = jnp.where(iota == idx, -1.0, x)
    tw = jnp.concatenate(vals, axis=1)
    ti = jnp.concatenate(ids, axis=1)
    tw_ref[...] = tw / jnp.sum(tw, axis=1, keepdims=True)
    ti_ref[...] = ti


RK = 128


def _rank_body(flat_ref, rank_ref):
    x = flat_ref[...]
    cpr = lax.broadcasted_iota(jnp.int32, (RK, RK), 0)
    cpc = lax.broadcasted_iota(jnp.int32, (RK, RK), 1)
    triu_incl = (cpr <= cpc).astype(jnp.bfloat16)
    tril_strict = (cpc < cpr).astype(jnp.bfloat16)
    occ_acc = jnp.zeros((RK, RK), dtype=jnp.float32)
    off_acc = jnp.zeros((RK, RK), dtype=jnp.float32)
    running = jnp.zeros((1, 1), dtype=jnp.float32)
    for e in range(NUM_EXPERTS):
        mask = (x == e)
        maskb = mask.astype(jnp.bfloat16)
        pref = lax.dot_general(maskb, triu_incl, (((1,), (0,)), ((), ())),
                               preferred_element_type=jnp.float32)
        rtot = pref[:, RK - 1:RK]
        rowoff = lax.dot_general(tril_strict, rtot.astype(jnp.bfloat16),
                                 (((1,), (0,)), ((), ())),
                                 preferred_element_type=jnp.float32)
        occ_e = pref + rowoff
        maskf = mask.astype(jnp.float32)
        occ_acc = occ_acc + maskf * occ_e
        off_acc = off_acc + maskf * running
        running = running + rowoff[RK - 1:RK, 0:1] + rtot[RK - 1:RK, 0:1]
    rank = occ_acc - 1.0 + off_acc
    rank_ref[...] = jnp.floor(rank + 0.5).astype(jnp.int32)


def _sc_scatter_body(rank_hbm, vals_hbm, out_hbm, idx_v, vals_v, sem):
    wid = lax.axis_index("s") * SC_NC + lax.axis_index("c")
    base = wid * ROWS_PER_W
    pltpu.sync_copy(rank_hbm.at[wid], idx_v)
    pltpu.sync_copy(vals_hbm.at[pl.ds(base, ROWS_PER_W)], vals_v)
    for j in range(ROWS_PER_W // 128):
        pltpu.async_copy(vals_v.at[pl.ds(j * 128, 128)],
                         out_hbm.at[idx_v.at[j]], sem).wait()


def _invert_rank_sc(rank3, vals):
    mesh = plsc.VectorSubcoreMesh(core_axis_name="c", subcore_axis_name="s")
    kfn = pl.kernel(
        _sc_scatter_body,
        out_type=jax.ShapeDtypeStruct((N_PERM, 128), jnp.int32),
        mesh=mesh,
        scratch_types=[
            pltpu.VMEM((ROWS_PER_W // 128, 128), jnp.int32),
            pltpu.VMEM((ROWS_PER_W, 128), jnp.int32),
            pltpu.SemaphoreType.DMA,
        ],
    )
    return kfn(rank3, vals)


def _sc_permute_body(out_hbm, rankt_hbm, perm_hbm, stage_v, idx_v, sem):
    wid = lax.axis_index("s") * SC_NC + lax.axis_index("c")
    pltpu.sync_copy(out_hbm.at[pl.ds(wid * (T // NW), T // NW)], stage_v)
    pltpu.sync_copy(rankt_hbm.at[wid], idx_v)
    handles = [pltpu.async_copy(stage_v, perm_hbm.at[idx_v.at[j]], sem)
               for j in range(TOP_K)]
    for h in handles:
        h.wait()


def _permute_scatter_sc(out, rankt3):
    mesh = plsc.VectorSubcoreMesh(core_axis_name="c", subcore_axis_name="s")
    kfn = pl.kernel(
        _sc_permute_body,
        out_type=jax.ShapeDtypeStruct((N_PERM, HIDDEN), jnp.float32),
        mesh=mesh,
        scratch_types=[
            pltpu.VMEM((T // NW, HIDDEN), jnp.float32),
            pltpu.VMEM((TOP_K, T // NW), jnp.int32),
            pltpu.SemaphoreType.DMA,
        ],
    )
    return kfn(out, rankt3)


def kernel(positions, hidden_states, kv_cache, w_pre, W_qkv, W_o, w_post, W_gate):
    pos2d = positions.astype(jnp.float32).reshape(T, 1)
    wpre2d = w_pre.reshape(1, HIDDEN)
    wpost2d = w_post.reshape(1, HIDDEN)

    q, k, v = pl.pallas_call(
        _qkv_body,
        grid=(T // TB,),
        in_specs=[
            pl.BlockSpec((TB, 1), lambda i: (i, 0)),
            pl.BlockSpec((TB, HIDDEN), lambda i: (i, 0)),
            pl.BlockSpec((1, HIDDEN), lambda i: (0, 0)),
            pl.BlockSpec((Q_SIZE + 2 * KV_SIZE, HIDDEN), lambda i: (0, 0)),
        ],
        out_specs=[
            pl.BlockSpec((TB, Q_SIZE), lambda i: (i, 0)),
            pl.BlockSpec((TB, KV_SIZE), lambda i: (i, 0)),
            pl.BlockSpec((TB, KV_SIZE), lambda i: (i, 0)),
        ],
        out_shape=[
            jax.ShapeDtypeStruct((T, Q_SIZE), jnp.bfloat16),
            jax.ShapeDtypeStruct((T, KV_SIZE), jnp.bfloat16),
            jax.ShapeDtypeStruct((T, KV_SIZE), jnp.bfloat16),
        ],
    )(pos2d, hidden_states, wpre2d, W_qkv.astype(jnp.bfloat16))

    attn = pl.pallas_call(
        _attn_body,
        grid=(T // QB,),
        in_specs=[
            pl.BlockSpec((QB, Q_SIZE), lambda qb: (qb, 0)),
            pl.BlockSpec((T, KV_SIZE), lambda qb: (0, 0)),
            pl.BlockSpec((T, KV_SIZE), lambda qb: (0, 0)),
        ],
        out_specs=pl.BlockSpec((QB, Q_SIZE), lambda qb: (qb, 0)),
        out_shape=jax.ShapeDtypeStruct((T, Q_SIZE), jnp.float32),
    )(q, k, v)

    out = pl.pallas_call(
        _oproj_body,
        grid=(T // TB,),
        in_specs=[
            pl.BlockSpec((TB, Q_SIZE), lambda i: (i, 0)),
            pl.BlockSpec((HIDDEN, Q_SIZE), lambda i: (0, 0)),
            pl.BlockSpec((TB, HIDDEN), lambda i: (i, 0)),
            pl.BlockSpec((1, HIDDEN), lambda i: (0, 0)),
        ],
        out_specs=pl.BlockSpec((TB, HIDDEN), lambda i: (i, 0)),
        out_shape=jax.ShapeDtypeStruct((T, HIDDEN), jnp.float32),
    )(attn, W_o.astype(jnp.bfloat16), hidden_states, wpost2d)

    key_r = jax.random.fold_in(jax.random.key(0), 123)
    expert_logits = jax.random.uniform(key_r, (T, NUM_EXPERTS), dtype=jnp.float32)
    topk_weights, topk_ids = pl.pallas_call(
        _topk_body,
        out_shape=[
            jax.ShapeDtypeStruct((T, TOP_K), jnp.float32),
            jax.ShapeDtypeStruct((T, TOP_K), jnp.int32),
        ],
    )(expert_logits)
    flat2 = topk_ids.reshape(RK, RK)
    rank2 = pl.pallas_call(
        _rank_body,
        out_shape=jax.ShapeDtypeStruct((RK, RK), jnp.int32),
    )(flat2)
    rank3 = rank2.reshape(NW, ROWS_PER_W // 128, 128)
    ii = jnp.arange(N_PERM, dtype=jnp.int32)
    vals = jnp.pad(ii[:, None], ((0, 0), (0, 127)))
    inv = _invert_rank_sc(rank3, vals)
    reorder_ids = inv[:, 0]
    rankt3 = rank2.reshape(NW, T // NW, TOP_K).transpose(0, 2, 1)
    permuted_output = _permute_scatter_sc(out, rankt3)
    return (permuted_output, topk_weights, topk_ids, reorder_ids)
